# baseline (speedup 1.0000x reference)
import functools

import jax
import jax.numpy as jnp
from jax import lax
from jax.experimental import pallas as pl
from jax.experimental.pallas import tpu as pltpu
from jax.experimental.pallas import tpu_sc as plsc

_B = 128
_D = 256
_SLOTS = 7
_SLOTS_PAD = 8
_ROWS = _B * _SLOTS_PAD
_NW = 16
_RPW = _ROWS // _NW
_H = 64
_W = 64
_BCHUNK = 8
_NDMA = _B // _BCHUNK
_NSEM = 8
_TWO_PI = 6.283185307179586


def _sc_gather(table, idx):
    mesh = plsc.VectorSubcoreMesh(core_axis_name="c", subcore_axis_name="s",
                                  num_cores=1, num_subcores=16)

    @functools.partial(
        pl.kernel,
        out_type=jax.ShapeDtypeStruct((_ROWS, _D), jnp.float32),
        mesh=mesh,
        scratch_types=[
            pltpu.VMEM((_RPW,), jnp.int32),
            pltpu.VMEM((_RPW, _D), jnp.float32),
            pltpu.SemaphoreType.DMA,
        ],
    )
    def k(table_hbm, idx_hbm, out_hbm, idx_v, rows_v, sem):
        wid = lax.axis_index("s")
        base = wid * _RPW
        pltpu.sync_copy(idx_hbm.at[pl.ds(base, _RPW)], idx_v)
        pltpu.async_copy(table_hbm.at[idx_v], rows_v, sem).wait()
        pltpu.sync_copy(rows_v, out_hbm.at[pl.ds(base, _RPW)])

    return k(table, idx)


def _dense_body(row_ref, dense_ref, plane, sem):
    x = row_ref[...][None, None, :, :]
    plane[...] = jnp.broadcast_to(x, (_BCHUNK, _H, _W, _D))
    for i in range(_NDMA):
        pltpu.make_async_copy(
            plane, dense_ref.at[pl.ds(i * _BCHUNK, _BCHUNK)],
            sem.at[i % _NSEM]).start()
    for i in range(_NDMA):
        pltpu.make_async_copy(
            plane, dense_ref.at[pl.ds(i * _BCHUNK, _BCHUNK)],
            sem.at[i % _NSEM]).wait()


def _dense_embed(row):
    return pl.pallas_call(
        _dense_body,
        out_shape=jax.ShapeDtypeStruct((_B, _H, _W, _D), jnp.float32),
        in_specs=[pl.BlockSpec(memory_space=pltpu.MemorySpace.VMEM)],
        out_specs=pl.BlockSpec(memory_space=pl.ANY),
        scratch_shapes=[
            pltpu.VMEM((_BCHUNK, _H, _W, _D), jnp.float32),
            pltpu.SemaphoreType.DMA((_NSEM,)),
        ],
    )(row)


def _pts_body(gat_ref, coords_ref, gauss_ref, pts_ref):
    c = coords_ref[...] * (1.0 / 512.0) - 1.0
    g0 = gauss_ref[0:1, :][None, :, :]
    g1 = gauss_ref[1:2, :][None, :, :]
    t = (c[:, :, 0:1] * g0 + c[:, :, 1:2] * g1) * _TWO_PI
    pos = jnp.concatenate([jnp.sin(t), jnp.cos(t)], axis=-1)
    pts_ref[0:2, :, :] = gat_ref[0:2, :, :] + pos
    pts_ref[2:_SLOTS, :, :] = gat_ref[2:_SLOTS, :, :]


def _pts_embed(gathered_sb, coords_sb, pe_gauss):
    return pl.pallas_call(
        _pts_body,
        out_shape=jax.ShapeDtypeStruct((_SLOTS, _B, _D), jnp.float32),
    )(gathered_sb, coords_sb, pe_gauss)


def kernel(points, point_labels, boxes, box_labels, label_table, pe_gauss):
    idx = jnp.concatenate(
        [point_labels[:, 0], box_labels[:, 0, 0],
         jnp.repeat(jnp.arange(6, 11, dtype=jnp.int32), _B),
         jnp.zeros((_B,), jnp.int32)])
    labels_sb = idx.reshape(_SLOTS_PAD, _B)
    coords_sb = jnp.stack(
        [points[:, 0, :], boxes[:, 0, 0, :]], axis=0)

    gathered = _sc_gather(label_table, idx)
    gathered_sb = gathered.reshape(_SLOTS_PAD, _B, _D)

    dense = _dense_embed(label_table[0:1, :])
    dense = jnp.transpose(dense, (0, 3, 1, 2))

    pts = _pts_embed(gathered_sb, coords_sb, pe_gauss)
    pts = jnp.transpose(pts, (1, 0, 2))

    pad = jnp.zeros((_B, _SLOTS), jnp.float32)
    ac = jnp.concatenate(
        [jnp.transpose(coords_sb, (1, 0, 2)),
         jnp.zeros((_B, _SLOTS - 2, 2), jnp.float32)], axis=1)
    al = jnp.transpose(labels_sb[:_SLOTS, :], (1, 0))
    return pts, dense, pad, ac, al


# speedup vs baseline: 1.0071x; 1.0071x over previous
import functools

import jax
import jax.numpy as jnp
from jax import lax
from jax.experimental import pallas as pl
from jax.experimental.pallas import tpu as pltpu
from jax.experimental.pallas import tpu_sc as plsc

_B = 128
_D = 256
_SLOTS = 7
_SLOTS_PAD = 8
_ROWS = _B * _SLOTS
_NW = 16
_RPW = _ROWS // _NW
_H = 64
_W = 64
_BCHUNK = 8
_NDMA = _B // _BCHUNK
_NSEM = 8
_TWO_PI = 6.283185307179586


def _sc_gather(table, idx):
    mesh = plsc.VectorSubcoreMesh(core_axis_name="c", subcore_axis_name="s",
                                  num_cores=1, num_subcores=16)

    @functools.partial(
        pl.kernel,
        out_type=jax.ShapeDtypeStruct((_ROWS, _D), jnp.float32),
        mesh=mesh,
        scratch_types=[
            pltpu.VMEM((_RPW,), jnp.int32),
            pltpu.VMEM((_RPW, _D), jnp.float32),
            pltpu.SemaphoreType.DMA,
        ],
    )
    def k(table_hbm, idx_hbm, out_hbm, idx_v, rows_v, sem):
        wid = lax.axis_index("s")
        base = wid * _RPW
        pltpu.sync_copy(idx_hbm.at[pl.ds(base, _RPW)], idx_v)
        pltpu.async_copy(table_hbm.at[idx_v], rows_v, sem).wait()
        pltpu.sync_copy(rows_v, out_hbm.at[pl.ds(base, _RPW)])

    return k(table, idx)


def _dense_body(row_ref, dense_ref, plane, sem):
    x = row_ref[...][None, None, :, :]
    plane[...] = jnp.broadcast_to(x, (_BCHUNK, _H, _W, _D))
    for i in range(_NDMA):
        pltpu.make_async_copy(
            plane, dense_ref.at[pl.ds(i * _BCHUNK, _BCHUNK)],
            sem.at[i % _NSEM]).start()
    for i in range(_NDMA):
        pltpu.make_async_copy(
            plane, dense_ref.at[pl.ds(i * _BCHUNK, _BCHUNK)],
            sem.at[i % _NSEM]).wait()


def _dense_embed(row):
    return pl.pallas_call(
        _dense_body,
        out_shape=jax.ShapeDtypeStruct((_B, _H, _W, _D), jnp.float32),
        in_specs=[pl.BlockSpec(memory_space=pltpu.MemorySpace.VMEM)],
        out_specs=pl.BlockSpec(memory_space=pl.ANY),
        scratch_shapes=[
            pltpu.VMEM((_BCHUNK, _H, _W, _D), jnp.float32),
            pltpu.SemaphoreType.DMA((_NSEM,)),
        ],
    )(row)


def _pts_body(gat_ref, lab_ref, coords_ref, gauss_ref,
              pts_ref, pad_ref, al_ref):
    c = coords_ref[...] * (1.0 / 512.0) - 1.0
    g0 = gauss_ref[0:1, :][None, :, :]
    g1 = gauss_ref[1:2, :][None, :, :]
    t = (c[:, :, 0:1] * g0 + c[:, :, 1:2] * g1) * _TWO_PI
    pos = jnp.concatenate([jnp.sin(t), jnp.cos(t)], axis=-1)
    pts_ref[0:2, :, :] = gat_ref[0:2, :, :] + pos
    pts_ref[2:_SLOTS, :, :] = gat_ref[2:_SLOTS, :, :]
    pad_ref[...] = jnp.zeros((_SLOTS, _B), jnp.float32)
    al_ref[...] = lab_ref[...]


def _pts_embed(gathered_sb, labels_sb, coords_sb, pe_gauss):
    return pl.pallas_call(
        _pts_body,
        out_shape=(
            jax.ShapeDtypeStruct((_SLOTS, _B, _D), jnp.float32),
            jax.ShapeDtypeStruct((_SLOTS, _B), jnp.float32),
            jax.ShapeDtypeStruct((_SLOTS, _B), jnp.int32),
        ),
    )(gathered_sb, labels_sb, coords_sb, pe_gauss)


def kernel(points, point_labels, boxes, box_labels, label_table, pe_gauss):
    idx = jnp.concatenate(
        [point_labels[:, 0], box_labels[:, 0, 0],
         jnp.repeat(jnp.arange(6, 11, dtype=jnp.int32), _B)])
    labels_sb = idx.reshape(_SLOTS, _B)
    coords_sb = jnp.stack(
        [points[:, 0, :], boxes[:, 0, 0, :]], axis=0)

    gathered = _sc_gather(label_table, idx)
    gathered_sb = gathered.reshape(_SLOTS, _B, _D)

    dense = _dense_embed(label_table[0:1, :])
    dense = jnp.transpose(dense, (0, 3, 1, 2))

    pts, pad_t, al_t = _pts_embed(gathered_sb, labels_sb, coords_sb,
                                  pe_gauss)
    pts = jnp.transpose(pts, (1, 0, 2))
    pad = jnp.transpose(pad_t, (1, 0))
    al = jnp.transpose(al_t, (1, 0))
    ac = jnp.concatenate(
        [jnp.transpose(coords_sb, (1, 0, 2)),
         jnp.zeros((_B, _SLOTS - 2, 2), jnp.float32)], axis=1)
    return pts, dense, pad, ac, al
